# baseline (speedup 1.0000x reference)
import jax
import jax.numpy as jnp
from jax.experimental import pallas as pl
from jax.experimental.pallas import tpu as pltpu
from jax.experimental.pallas import tpu_sc as plsc

_WINDOW = 256
_CHUNKS = 4
_TC_BLOCK = 1024


def _sc_gather(padded, indices, chunk_n):
    mesh = plsc.VectorSubcoreMesh(core_axis_name="core",
                                  subcore_axis_name="subcore")

    @pl.kernel(
        out_type=jax.ShapeDtypeStruct((chunk_n, 128), padded.dtype),
        mesh=mesh,
    )
    def gather(tab_hbm, idx_hbm, out_hbm):
        def body(idx_vmem, out_vmem):
            pltpu.sync_copy(tab_hbm.at[idx_vmem.at[0]], out_vmem)

        pltpu.emit_pipeline(
            body,
            grid=(chunk_n // _WINDOW,),
            in_specs=[pl.BlockSpec((1, _WINDOW),
                                   index_map=lambda i: (0, i))],
            out_specs=[pl.BlockSpec((_WINDOW, 128),
                                    index_map=lambda i: (i, 0))],
            core_axis_name=("core", "subcore"),
            dimension_semantics=(pltpu.PARALLEL,),
        )(idx_hbm, out_hbm)

    return gather(padded, indices)


def _tc_slice(x, dim):
    n = x.shape[0]

    def body(x_ref, o_ref):
        o_ref[...] = x_ref[:, :dim]

    return pl.pallas_call(
        body,
        grid=(n // _TC_BLOCK,),
        in_specs=[pl.BlockSpec((_TC_BLOCK, 128), lambda i: (i, 0))],
        out_specs=pl.BlockSpec((_TC_BLOCK, dim), lambda i: (i, 0)),
        out_shape=jax.ShapeDtypeStruct((n, dim), x.dtype),
    )(x)


def kernel(model_input, table):
    batch, seq = model_input.shape
    num_idx = batch * seq
    rows, dim = table.shape
    flat_idx = model_input.reshape(num_idx)
    chunk_n = num_idx // _CHUNKS

    padded = jnp.pad(table, ((0, 0), (0, 128 - dim)))

    outs = []
    for c in range(_CHUNKS):
        idx_c = jax.lax.dynamic_slice(flat_idx, (c * chunk_n,), (chunk_n,))
        g = _sc_gather(padded, idx_c.reshape(1, chunk_n), chunk_n)
        outs.append(_tc_slice(g, dim))

    out = jnp.concatenate(outs, axis=0)
    return out.reshape(batch, seq, dim)


# speedup vs baseline: 1.2942x; 1.2942x over previous
import jax
import jax.numpy as jnp
from jax import lax
from jax.experimental import pallas as pl
from jax.experimental.pallas import tpu as pltpu
from jax.experimental.pallas import tpu_sc as plsc

_WINDOW = 256
_NBUF = 2
_NUM_WORKERS = 32
_TC_BLOCK = 2048


def _sc_gather(padded, indices, num_idx):
    per_worker = num_idx // _NUM_WORKERS
    steps = per_worker // (_WINDOW * _NBUF)

    mesh = plsc.VectorSubcoreMesh(core_axis_name="core",
                                  subcore_axis_name="subcore")

    @pl.kernel(
        out_type=jax.ShapeDtypeStruct((num_idx, 128), padded.dtype),
        mesh=mesh,
        scratch_types=(
            [pltpu.VMEM((_WINDOW,), jnp.int32)] * _NBUF
            + [pltpu.VMEM((_WINDOW, 128), jnp.float32)] * _NBUF
            + [pltpu.SemaphoreType.DMA((_NBUF,)),
               pltpu.SemaphoreType.DMA((_NBUF,))]
        ),
    )
    def gather(tab_hbm, idx_hbm, out_hbm, *scratch):
        idx_v = scratch[:_NBUF]
        rows_v = scratch[_NBUF:2 * _NBUF]
        gsem, osem = scratch[2 * _NBUF], scratch[2 * _NBUF + 1]
        wid = lax.axis_index("subcore") * 2 + lax.axis_index("core")
        base = wid * per_worker

        def start(step, buf):
            lo = base + (step * _NBUF + buf) * _WINDOW
            pltpu.sync_copy(idx_hbm.at[pl.ds(lo, _WINDOW)], idx_v[buf])
            pltpu.make_async_copy(
                tab_hbm.at[idx_v[buf]], rows_v[buf], gsem.at[buf]
            ).start()

        def drain(step, buf):
            lo = base + (step * _NBUF + buf) * _WINDOW
            pltpu.make_async_copy(
                tab_hbm.at[idx_v[buf]], rows_v[buf], gsem.at[buf]
            ).wait()
            pltpu.make_async_copy(
                rows_v[buf], out_hbm.at[pl.ds(lo, _WINDOW)], osem.at[buf]
            ).start()

        def wait_out(step, buf):
            lo = base + (step * _NBUF + buf) * _WINDOW
            pltpu.make_async_copy(
                rows_v[buf], out_hbm.at[pl.ds(lo, _WINDOW)], osem.at[buf]
            ).wait()

        for b in range(_NBUF):
            start(0, b)

        @pl.loop(0, steps - 1)
        def _(s):
            for b in range(_NBUF):
                drain(s, b)
                wait_out(s, b)
                start(s + 1, b)

        for b in range(_NBUF):
            drain(steps - 1, b)
            wait_out(steps - 1, b)

    return gather(padded, indices)


def _tc_slice(x, dim, num_idx):
    def body(x_ref, o_ref):
        o_ref[...] = x_ref[:, :dim]

    return pl.pallas_call(
        body,
        grid=(num_idx // _TC_BLOCK,),
        in_specs=[pl.BlockSpec((_TC_BLOCK, 128), lambda i: (i, 0))],
        out_specs=pl.BlockSpec((_TC_BLOCK, dim), lambda i: (i, 0)),
        out_shape=jax.ShapeDtypeStruct((num_idx, dim), x.dtype),
    )(x)


def kernel(model_input, table):
    batch, seq = model_input.shape
    num_idx = batch * seq
    rows, dim = table.shape
    indices = model_input.reshape(num_idx)

    padded = jnp.pad(table, ((0, 0), (0, 128 - dim)))

    g = _sc_gather(padded, indices, num_idx)
    out = _tc_slice(g, dim, num_idx)
    return out.reshape(batch, seq, dim)


# speedup vs baseline: 1.8054x; 1.3951x over previous
import jax
import jax.numpy as jnp
from jax.experimental import pallas as pl
from jax.experimental.pallas import tpu as pltpu
from jax.experimental.pallas import tpu_sc as plsc

_WINDOW = 256


def kernel(model_input, table):
    batch, seq = model_input.shape
    num_idx = batch * seq
    rows, dim = table.shape
    indices = model_input.reshape(1, num_idx)

    padded = jnp.pad(table, ((0, 0), (0, 128 - dim)))

    mesh = plsc.VectorSubcoreMesh(core_axis_name="core",
                                  subcore_axis_name="subcore")

    @pl.kernel(
        out_type=jax.ShapeDtypeStruct((num_idx, 128), table.dtype),
        mesh=mesh,
    )
    def gather(tab_hbm, idx_hbm, out_hbm):
        def body(idx_vmem, out_vmem):
            pltpu.sync_copy(tab_hbm.at[idx_vmem.at[0]], out_vmem)

        pltpu.emit_pipeline(
            body,
            grid=(num_idx // _WINDOW,),
            in_specs=[pl.BlockSpec((1, _WINDOW),
                                   index_map=lambda i: (0, i))],
            out_specs=[pl.BlockSpec((_WINDOW, 128),
                                    index_map=lambda i: (i, 0))],
            core_axis_name=("core", "subcore"),
            dimension_semantics=(pltpu.PARALLEL,),
        )(idx_hbm, out_hbm)

    out = gather(padded, indices)
    return out[:, :dim].reshape(batch, seq, dim)
